# Initial kernel scaffold; baseline (speedup 1.0000x reference)
import jax
import jax.numpy as jnp
from jax.experimental import pallas as pl


def kernel(x, table):
    raise NotImplementedError("write your pallas kernel here")


# speedup vs baseline: 69.3063x; 69.3063x over previous
import functools

import jax
import jax.numpy as jnp
from jax.experimental import pallas as pl

_R = 16
_NV = 2 * _R + 1


def _pe_add_body(x_ref, t_ref, o_ref, *, sb, s_total):
    j0 = pl.program_id(0) * sb
    jj = jax.lax.broadcasted_iota(jnp.int32, (sb, _NV), 0) + j0
    vv = jax.lax.broadcasted_iota(jnp.int32, (sb, _NV), 1)
    row = jj - vv + _R
    interior = ((row >= 0) & (row < s_total)).astype(jnp.float32)
    c_lo = jnp.maximum(s_total - _R - jj, 0).astype(jnp.float32)
    c_hi = jnp.maximum(jj - (_R - 1), 0).astype(jnp.float32)
    counts = jnp.where(vv == 0, c_lo, jnp.where(vv == _NV - 1, c_hi, interior))
    pe = jnp.dot(counts, t_ref[...], preferred_element_type=jnp.float32)
    pe = pe * (1.0 / s_total)
    o_ref[...] = x_ref[...] + pe[None, :, :]


def kernel(x, table):
    B, S, D = x.shape
    sb = 512
    body = functools.partial(_pe_add_body, sb=sb, s_total=S)
    return pl.pallas_call(
        body,
        grid=(S // sb,),
        in_specs=[
            pl.BlockSpec((B, sb, D), lambda i: (0, i, 0)),
            pl.BlockSpec((_NV, D), lambda i: (0, 0)),
        ],
        out_specs=pl.BlockSpec((B, sb, D), lambda i: (0, i, 0)),
        out_shape=jax.ShapeDtypeStruct(x.shape, x.dtype),
    )(x, table)
